# baseline (speedup 1.0000x reference)
import functools

import jax
import jax.numpy as jnp
from jax import lax
from jax.experimental import pallas as pl
from jax.experimental.pallas import tpu as pltpu
from jax.experimental.pallas import tpu_sc as plsc

NC = 2
NS = 16
NW = NC * NS
L = 16
B_A = 64
B_B = 64
DEPTH = 2


def _sc_mesh():
    return plsc.VectorSubcoreMesh(
        core_axis_name="c", subcore_axis_name="s", num_cores=NC,
        num_subcores=NS)


def _make_pass(n_pad, ch, d, bsz, mode):
    rps = n_pad // NS
    ustep = 6
    assert ch % ustep == 0

    scratch = [
        pltpu.VMEM((2, bsz), jnp.int32),
        pltpu.VMEM((2, bsz), jnp.int32),
        pltpu.VMEM((2, bsz), jnp.int32),
        pltpu.VMEM((bsz, d), jnp.float32),
        pltpu.VMEM((bsz, d), jnp.float32),
        pltpu.VMEM((n_pad,), jnp.float32),
    ]
    if mode == 'b':
        scratch.append(pltpu.VMEM((n_pad,), jnp.float32))
    scratch += [
        pltpu.VMEM_SHARED((n_pad, d), jnp.float32),
        pltpu.SemaphoreType.DMA,
        pltpu.SemaphoreType.DMA,
        pltpu.SemaphoreType.DMA,
    ]

    def body(*refs):
        if mode == 'a':
            (x_hbm, edges_hbm, zero_d_hbm, zero_1_hbm,
             sums_out, vec_out) = refs[:6]
            scr = refs[6:]
        else:
            (x_hbm, inv_hbm, edges_hbm, zero_d_hbm, zero_1_hbm,
             sums_out, vec_out) = refs[:7]
            scr = refs[7:]
        idxb = [scr[0], scr[1], scr[2]]
        rows = [scr[3], scr[4]]
        vec_v = scr[5]
        pos = 6
        if mode == 'b':
            inv_v = scr[pos]
            pos += 1
        sum_acc = scr[pos]
        sems = [scr[pos + 1], scr[pos + 2]]
        sem_i = scr[pos + 3]

        c = lax.axis_index("c")
        s = lax.axis_index("s")
        wid = s * NC + c
        stripe = pl.ds(s * rps, rps)

        pltpu.sync_copy(zero_d_hbm.at[stripe], sum_acc.at[stripe])
        pltpu.sync_copy(zero_1_hbm, vec_v)
        if mode == 'b':
            pltpu.sync_copy(inv_hbm, inv_v)
        plsc.subcore_barrier()

        ones = jnp.ones((L,), jnp.float32)
        dummy = zero_d_hbm.at[pl.ds(0, bsz)]
        dummy_i = edges_hbm.at[wid, 0]

        def regop(t3):
            def grp(g, _):
                dv = idxb[t3][1, pl.ds(g * L, L)]
                if mode == 'a':
                    plsc.addupdate_scatter(vec_v, [dv], ones)
                else:
                    sv = idxb[t3][0, pl.ds(g * L, L)]
                    vals = plsc.load_gather(inv_v, [dv])
                    plsc.addupdate_scatter(vec_v, [sv], vals)
                return 0
            lax.fori_loop(0, bsz // L, grp, 0)

        pltpu.sync_copy(edges_hbm.at[wid, 0], idxb[0])
        pltpu.async_copy(x_hbm.at[idxb[0].at[0]], rows[0], sems[0])
        pltpu.async_copy(edges_hbm.at[wid, 1], idxb[1], sem_i)

        def block(q, _):
            for t in range(ustep):
                k = q * ustep + t
                t3 = t % 3
                r2 = t % 2
                pltpu.make_async_copy(dummy_i, idxb[(t3 + 1) % 3],
                                      sem_i).wait()
                pltpu.async_copy(x_hbm.at[idxb[(t3 + 1) % 3].at[0]],
                                 rows[1 - r2], sems[1 - r2])
                pltpu.async_copy(edges_hbm.at[wid, lax.rem(k + 2, ch)],
                                 idxb[(t3 + 2) % 3], sem_i)
                regop(t3)
                pltpu.make_async_copy(dummy, rows[r2], sems[r2]).wait()
                pltpu.sync_copy(rows[r2], sum_acc.at[idxb[t3].at[1]],
                                add=True)
            return 0
        lax.fori_loop(0, ch // ustep, block, 0)
        pltpu.make_async_copy(dummy, rows[0], sems[0]).wait()
        pltpu.make_async_copy(dummy_i, idxb[1], sem_i).wait()

        plsc.subcore_barrier()
        pltpu.sync_copy(sum_acc.at[stripe], sums_out.at[c, stripe])
        pltpu.sync_copy(vec_v, vec_out.at[wid])

    return pl.kernel(
        body,
        out_type=[
            jax.ShapeDtypeStruct((NC, n_pad, d), jnp.float32),
            jax.ShapeDtypeStruct((NW, n_pad), jnp.float32),
        ],
        mesh=_sc_mesh(),
        compiler_params=pltpu.CompilerParams(needs_layout_passes=False),
        scratch_types=scratch,
    )


def _layer_body(make_inv, n, n_pad,
                x_ref, sums_ref, degs_ref, ws_ref, wn_ref, b_ref, g_ref,
                be_ref, *out_refs):
    x = x_ref[...]
    summed = sums_ref[0, :n, :] + sums_ref[1, :n, :]
    deg_full = jnp.sum(degs_ref[...], axis=0)
    deg = deg_full[:n, None]
    agg = jnp.where(deg > 0, summed / jnp.maximum(deg, 1.0), 0.0)
    t = (jnp.dot(x, ws_ref[...], preferred_element_type=jnp.float32)
         + jnp.dot(agg, wn_ref[...], preferred_element_type=jnp.float32)
         + b_ref[...])
    m = jnp.mean(t, axis=0, keepdims=True)
    v = jnp.mean(jnp.square(t - m), axis=0, keepdims=True)
    h = g_ref[...] * (t - m) * lax.rsqrt(v + 1e-5) + be_ref[...]
    out_refs[0][...] = jnp.maximum(h, 0.0)
    if make_inv:
        row = lax.iota(jnp.int32, n_pad)
        inv = jnp.where(row < n, 1.0 / jnp.maximum(deg_full, 1.0), 0.0)
        out_refs[1][...] = inv


def _final_body(n, feat_ref, cv_ref, ws_ref, wn_ref, b_ref, out_ref):
    feat = feat_ref[...]
    cvec = jnp.sum(cv_ref[...], axis=0)[:n, None]
    sacc = jnp.sum(feat * cvec, axis=0, keepdims=True)
    mf = jnp.mean(feat, axis=0, keepdims=True)
    o = (jnp.dot(mf, ws_ref[...], preferred_element_type=jnp.float32)
         + jnp.dot(sacc / n, wn_ref[...], preferred_element_type=jnp.float32)
         + b_ref[...])
    z = o - jnp.max(o, axis=-1, keepdims=True)
    out_ref[...] = z - jnp.log(jnp.sum(jnp.exp(z), axis=-1, keepdims=True))


def _pad_edges(edge_index, n, e, bsz):
    ch = -(-e // (NW * bsz))
    ch += (-ch) % 6
    e_pad = ch * NW * bsz
    src = edge_index[0]
    dst = edge_index[1]
    pad = e_pad - e
    if pad:
        src = jnp.concatenate([src, jnp.zeros((pad,), jnp.int32)])
        dst = jnp.concatenate([dst, jnp.full((pad,), n, jnp.int32)])
    edges = jnp.stack([src.reshape(NW, ch, bsz),
                       dst.reshape(NW, ch, bsz)], axis=2)
    return edges, ch


def kernel(edge_index, inputs, W_self0, W_neigh0, b0, gamma0, beta0,
           W_self1, W_neigh1, b1, gamma1, beta1, W_self2, W_neigh2, b2):
    n, d = inputs.shape
    e = edge_index.shape[1]
    d_out = W_self2.shape[1]

    n_pad = -(-(n + 1) // (NS * 8)) * (NS * 8)
    edges_a, ch_a = _pad_edges(edge_index, n, e, B_A)
    edges_b, ch_b = _pad_edges(edge_index, n, e, B_B)
    zero_d = jnp.zeros((n_pad, d), jnp.float32)
    zero_1 = jnp.zeros((n_pad,), jnp.float32)

    pass_a = _make_pass(n_pad, ch_a, d, B_A, 'a')
    pass_b = _make_pass(n_pad, ch_b, d, B_B, 'b')

    def layer(x, sums, degs, ws, wn, b, g, be, make_inv):
        outs = [jax.ShapeDtypeStruct((n, d), jnp.float32)]
        if make_inv:
            outs.append(jax.ShapeDtypeStruct((n_pad,), jnp.float32))
        return pl.pallas_call(
            functools.partial(_layer_body, make_inv, n, n_pad),
            out_shape=outs,
        )(x, sums, degs, ws, wn, b, g, be)

    sums_a, degv = pass_a(inputs, edges_a, zero_d, zero_1)
    h1, invd = layer(inputs, sums_a, degv, W_self0, W_neigh0, b0, gamma0,
                     beta0, True)
    sums_b, cv = pass_b(h1, invd, edges_b, zero_d, zero_1)
    (feat,) = layer(h1, sums_b, degv, W_self1, W_neigh1, b1, gamma1,
                    beta1, False)
    out = pl.pallas_call(
        functools.partial(_final_body, n),
        out_shape=jax.ShapeDtypeStruct((1, d_out), jnp.float32),
    )(feat, cv, W_self2, W_neigh2, b2)
    return out, inputs, feat


# speedup vs baseline: 1.9641x; 1.9641x over previous
import functools

import jax
import jax.numpy as jnp
from jax import lax
from jax.experimental import pallas as pl
from jax.experimental.pallas import tpu as pltpu
from jax.experimental.pallas import tpu_sc as plsc

NC = 2
NS = 16
NW = NC * NS
L = 16
B_A = 64
B_B = 64
DEPTH = 2


def _sc_mesh():
    return plsc.VectorSubcoreMesh(
        core_axis_name="c", subcore_axis_name="s", num_cores=NC,
        num_subcores=NS)


def _make_pass(n_pad, ch, d, bsz, mode):
    rps = n_pad // NS
    dep = 2
    assert ch % dep == 0

    scratch = [
        pltpu.VMEM((2, bsz), jnp.int32),
        pltpu.VMEM((2, bsz), jnp.int32),
        pltpu.VMEM((bsz, d), jnp.float32),
        pltpu.VMEM((bsz, d), jnp.float32),
        pltpu.VMEM((n_pad,), jnp.float32),
    ]
    if mode == 'b':
        scratch.append(pltpu.VMEM((n_pad,), jnp.float32))
    scratch += [
        pltpu.VMEM_SHARED((n_pad, d), jnp.float32),
        pltpu.SemaphoreType.DMA,
        pltpu.SemaphoreType.DMA,
    ]

    def body(*refs):
        if mode == 'a':
            (x_hbm, edges_hbm, zero_d_hbm, zero_1_hbm,
             sums_out, vec_out) = refs[:6]
            scr = refs[6:]
        else:
            (x_hbm, inv_hbm, edges_hbm, zero_d_hbm, zero_1_hbm,
             sums_out, vec_out) = refs[:7]
            scr = refs[7:]
        idxb = [scr[0], scr[1]]
        rows = [scr[2], scr[3]]
        vec_v = scr[4]
        pos = 5
        if mode == 'b':
            inv_v = scr[pos]
            pos += 1
        sum_acc = scr[pos]
        sems = [scr[pos + 1], scr[pos + 2]]

        c = lax.axis_index("c")
        s = lax.axis_index("s")
        wid = s * NC + c
        stripe = pl.ds(s * rps, rps)

        pltpu.sync_copy(zero_d_hbm.at[stripe], sum_acc.at[stripe])
        pltpu.sync_copy(zero_1_hbm, vec_v)
        if mode == 'b':
            pltpu.sync_copy(inv_hbm, inv_v)
        plsc.subcore_barrier()

        ones = jnp.ones((L,), jnp.float32)
        dummy = zero_d_hbm.at[pl.ds(0, bsz)]

        def regop(t):
            def grp(g, _):
                dv = idxb[t][1, pl.ds(g * L, L)]
                if mode == 'a':
                    plsc.addupdate_scatter(vec_v, [dv], ones)
                else:
                    sv = idxb[t][0, pl.ds(g * L, L)]
                    vals = plsc.load_gather(inv_v, [dv])
                    plsc.addupdate_scatter(vec_v, [sv], vals)
                return 0
            lax.fori_loop(0, bsz // L, grp, 0)

        def fetch(t, k):
            pltpu.sync_copy(edges_hbm.at[wid, k], idxb[t])
            pltpu.async_copy(x_hbm.at[idxb[t].at[0]], rows[t], sems[t])

        fetch(0, 0)

        def block(q, _):
            for t in range(dep):
                k = q * dep + t
                fetch((t + 1) % dep, lax.rem(k + 1, ch))
                regop(t)
                pltpu.make_async_copy(dummy, rows[t], sems[t]).wait()
                pltpu.sync_copy(rows[t], sum_acc.at[idxb[t].at[1]],
                                add=True)
            return 0
        lax.fori_loop(0, ch // dep, block, 0)
        pltpu.make_async_copy(dummy, rows[0], sems[0]).wait()

        plsc.subcore_barrier()
        pltpu.sync_copy(sum_acc.at[stripe], sums_out.at[c, stripe])
        pltpu.sync_copy(vec_v, vec_out.at[wid])

    return pl.kernel(
        body,
        out_type=[
            jax.ShapeDtypeStruct((NC, n_pad, d), jnp.float32),
            jax.ShapeDtypeStruct((NW, n_pad), jnp.float32),
        ],
        mesh=_sc_mesh(),
        compiler_params=pltpu.CompilerParams(needs_layout_passes=False),
        scratch_types=scratch,
    )


def _layer_body(make_inv, n, n_pad,
                x_ref, sums_ref, degs_ref, ws_ref, wn_ref, b_ref, g_ref,
                be_ref, *out_refs):
    x = x_ref[...]
    summed = sums_ref[0, :n, :] + sums_ref[1, :n, :]
    deg_full = jnp.sum(degs_ref[...], axis=0)
    deg = deg_full[:n, None]
    agg = jnp.where(deg > 0, summed / jnp.maximum(deg, 1.0), 0.0)
    t = (jnp.dot(x, ws_ref[...], preferred_element_type=jnp.float32)
         + jnp.dot(agg, wn_ref[...], preferred_element_type=jnp.float32)
         + b_ref[...])
    m = jnp.mean(t, axis=0, keepdims=True)
    v = jnp.mean(jnp.square(t - m), axis=0, keepdims=True)
    h = g_ref[...] * (t - m) * lax.rsqrt(v + 1e-5) + be_ref[...]
    out_refs[0][...] = jnp.maximum(h, 0.0)
    if make_inv:
        row = lax.iota(jnp.int32, n_pad)
        inv = jnp.where(row < n, 1.0 / jnp.maximum(deg_full, 1.0), 0.0)
        out_refs[1][...] = inv


def _final_body(n, feat_ref, cv_ref, ws_ref, wn_ref, b_ref, out_ref):
    feat = feat_ref[...]
    cvec = jnp.sum(cv_ref[...], axis=0)[:n, None]
    sacc = jnp.sum(feat * cvec, axis=0, keepdims=True)
    mf = jnp.mean(feat, axis=0, keepdims=True)
    o = (jnp.dot(mf, ws_ref[...], preferred_element_type=jnp.float32)
         + jnp.dot(sacc / n, wn_ref[...], preferred_element_type=jnp.float32)
         + b_ref[...])
    z = o - jnp.max(o, axis=-1, keepdims=True)
    out_ref[...] = z - jnp.log(jnp.sum(jnp.exp(z), axis=-1, keepdims=True))


def _pad_edges(edge_index, n, e, bsz):
    ch = -(-e // (NW * bsz))
    ch += (-ch) % 2
    e_pad = ch * NW * bsz
    src = edge_index[0]
    dst = edge_index[1]
    pad = e_pad - e
    if pad:
        src = jnp.concatenate([src, jnp.zeros((pad,), jnp.int32)])
        dst = jnp.concatenate([dst, jnp.full((pad,), n, jnp.int32)])
    edges = jnp.stack([src.reshape(NW, ch, bsz),
                       dst.reshape(NW, ch, bsz)], axis=2)
    return edges, ch


def kernel(edge_index, inputs, W_self0, W_neigh0, b0, gamma0, beta0,
           W_self1, W_neigh1, b1, gamma1, beta1, W_self2, W_neigh2, b2):
    n, d = inputs.shape
    e = edge_index.shape[1]
    d_out = W_self2.shape[1]

    n_pad = -(-(n + 1) // (NS * 8)) * (NS * 8)
    edges_a, ch_a = _pad_edges(edge_index, n, e, B_A)
    edges_b, ch_b = _pad_edges(edge_index, n, e, B_B)
    zero_d = jnp.zeros((n_pad, d), jnp.float32)
    zero_1 = jnp.zeros((n_pad,), jnp.float32)

    pass_a = _make_pass(n_pad, ch_a, d, B_A, 'a')
    pass_b = _make_pass(n_pad, ch_b, d, B_B, 'b')

    def layer(x, sums, degs, ws, wn, b, g, be, make_inv):
        outs = [jax.ShapeDtypeStruct((n, d), jnp.float32)]
        if make_inv:
            outs.append(jax.ShapeDtypeStruct((n_pad,), jnp.float32))
        return pl.pallas_call(
            functools.partial(_layer_body, make_inv, n, n_pad),
            out_shape=outs,
        )(x, sums, degs, ws, wn, b, g, be)

    sums_a, degv = pass_a(inputs, edges_a, zero_d, zero_1)
    h1, invd = layer(inputs, sums_a, degv, W_self0, W_neigh0, b0, gamma0,
                     beta0, True)
    sums_b, cv = pass_b(h1, invd, edges_b, zero_d, zero_1)
    (feat,) = layer(h1, sums_b, degv, W_self1, W_neigh1, b1, gamma1,
                    beta1, False)
    out = pl.pallas_call(
        functools.partial(_final_body, n),
        out_shape=jax.ShapeDtypeStruct((1, d_out), jnp.float32),
    )(feat, cv, W_self2, W_neigh2, b2)
    return out, inputs, feat
